# baseline (speedup 1.0000x reference)
import functools

import jax
import jax.numpy as jnp
from jax import lax
from jax.experimental import pallas as pl
from jax.experimental.pallas import tpu as pltpu
from jax.experimental.pallas import tpu_sc as plsc

N_EMB = 131072
C = 1024
D_UP = 2048
K_MAX = 4
EPS = 1.1920928955078125e-07

TOKENS = 4096
NC, NS = 2, 16
NW = NC * NS
TPW = TOKENS // NW
G = 8
NGRP = TPW // G
BOUNDS_PAD = 50016

_sc_mesh = plsc.VectorSubcoreMesh(core_axis_name="c", subcore_axis_name="s")


def _bcast(s):
    return jnp.broadcast_to(s, (16,))


def _vperm(v, idx):
    return lax.gather(
        v, idx[:, None],
        lax.GatherDimensionNumbers(offset_dims=(), collapsed_slice_dims=(0,),
                                   start_index_map=(0,)),
        slice_sizes=(1,), mode=lax.GatherScatterMode.PROMISE_IN_BOUNDS)


def _lane_sum(v):
    lanes = lax.iota(jnp.int32, 16)
    for d in (8, 4, 2, 1):
        v = v + _vperm(v, lanes ^ d)
    return v


def _lane_bcast(v, lane):
    return _vperm(v, lax.iota(jnp.int32, 16) * 0 + lane)


@functools.partial(
    pl.kernel,
    mesh=_sc_mesh,
    out_type=jax.ShapeDtypeStruct((TOKENS, C), jnp.float32),
    scratch_types=[
        pltpu.VMEM((TPW,), jnp.int32),
        pltpu.VMEM((TPW,), jnp.int32),
        pltpu.VMEM((TPW,), jnp.int32),
        pltpu.VMEM((TPW,), jnp.int32),
        pltpu.VMEM((K_MAX, TPW), jnp.int32),
        pltpu.VMEM((TPW,), jnp.int32),
        pltpu.VMEM((2, G, C), jnp.float32),
        pltpu.VMEM((2, K_MAX * G, C), jnp.float32),
        pltpu.VMEM((2, G, C), jnp.float32),
        pltpu.VMEM((8, 16), jnp.float32),
        pltpu.SemaphoreType.DMA,
        pltpu.SemaphoreType.DMA,
        pltpu.SemaphoreType.DMA,
        pltpu.SemaphoreType.DMA,
        pltpu.SemaphoreType.DMA,
        pltpu.SemaphoreType.DMA,
        pltpu.SemaphoreType.DMA,
    ],
)
def _sc_attn(ids_hbm, bounds_hbm, x_hbm, kv_hbm, agg_out,
             ids_v, ids1_v, starts_v, ends_v, idx_v, len_v,
             x_b, kv_b, ag_b, acc_st, sem, xs0, xs1, gs0, gs1, ws0, ws1):
    wid = lax.axis_index("s") * NC + lax.axis_index("c")
    base = wid * TPW
    xs = (xs0, xs1)
    gs = (gs0, gs1)
    ws = (ws0, ws1)

    pltpu.sync_copy(ids_hbm.at[pl.ds(base, TPW)], ids_v)
    for j in range(TPW // 16):
        ids1_v[pl.ds(j * 16, 16)] = ids_v[pl.ds(j * 16, 16)] + 1
    cs = pltpu.async_copy(bounds_hbm.at[ids_v], starts_v, sem)
    ce = pltpu.async_copy(bounds_hbm.at[ids1_v], ends_v, sem)
    cs.wait()
    ce.wait()
    for j in range(TPW // 16):
        s16 = starts_v[pl.ds(j * 16, 16)]
        e16 = ends_v[pl.ds(j * 16, 16)]
        len_v[pl.ds(j * 16, 16)] = e16 - s16
        for k in range(K_MAX):
            idx_v[k, pl.ds(j * 16, 16)] = jnp.minimum(s16 + k, N_EMB - 1)

    zv16 = jnp.zeros((16,), jnp.float32)
    for r in range(5):
        acc_st[r] = zv16

    def issue_group(g, b):
        pltpu.async_copy(x_hbm.at[pl.ds(base + g * G, G)], x_b.at[b], xs[b])
        for k in range(K_MAX):
            pltpu.async_copy(
                kv_hbm.at[idx_v.at[k, pl.ds(g * G, G)]],
                kv_b.at[b, pl.ds(k * G, G)], gs[b])

    def wait_group(g, b):
        pltpu.make_async_copy(
            x_hbm.at[pl.ds(base + g * G, G)], x_b.at[b], xs[b]).wait()
        for k in range(K_MAX):
            pltpu.make_async_copy(
                kv_hbm.at[idx_v.at[k, pl.ds(g * G, G)]],
                kv_b.at[b, pl.ds(k * G, G)], gs[b]).wait()

    def compute_group(g, b, lwinf, lwin_i):
        for t in range(G):
            ls = lwin_i[b * G + t]

            def make_dot(nk):
                def body(ci, accs):
                    xv = x_b[b, t, pl.ds(ci * 16, 16)]
                    out = [accs[k] + kv_b[b, k * G + t, pl.ds(ci * 16, 16)] * xv
                           for k in range(nk)]
                    out.append(accs[nk] + xv * xv)
                    return tuple(out)
                return body

            for v, cond in ((2, ls <= 2), (4, ls >= 3)):
                @pl.when(cond)
                def _(v=v):
                    z = jnp.zeros((16,), jnp.float32)
                    accs = lax.fori_loop(0, C // 16, make_dot(v),
                                         (z,) * (v + 1), unroll=4)
                    for k in range(v):
                        acc_st[k] = accs[k]
                    acc_st[4] = accs[v]

            sx = _lane_sum(acc_st[4])
            av = sx * (1.0 / C) + EPS
            ii = lax.bitcast_convert_type(av, jnp.int32)
            ii = jnp.int32(0x5F3759DF) - (ii >> 1)
            y = lax.bitcast_convert_type(ii, jnp.float32)
            for _ in range(4):
                y = y * (1.5 - 0.5 * av * y * y)
            s0 = _lane_sum(acc_st[0]) * y
            s1 = _lane_sum(acc_st[1]) * y
            s2 = _lane_sum(acc_st[2]) * y
            s3 = _lane_sum(acc_st[3]) * y
            lvf = _lane_bcast(lwinf, b * G + t)
            one = jnp.ones((16,), jnp.float32)
            zero = jnp.zeros((16,), jnp.float32)
            BIG = jnp.float32(1e30)
            v1 = jnp.minimum(jnp.maximum(lvf - 1.0, zero), one)
            v2 = jnp.minimum(jnp.maximum(lvf - 2.0, zero), one)
            v3 = jnp.minimum(jnp.maximum(lvf - 3.0, zero), one)
            sm1 = s1 + (v1 - 1.0) * BIG
            sm2 = s2 + (v2 - 1.0) * BIG
            sm3 = s3 + (v3 - 1.0) * BIG
            m = jnp.maximum(jnp.maximum(s0, sm1), jnp.maximum(sm2, sm3))
            e0 = jnp.exp(s0 - m)
            e1 = jnp.exp(sm1 - m)
            e2 = jnp.exp(sm2 - m)
            e3 = jnp.exp(sm3 - m)
            inv = 1.0 / (e0 + e1 + e2 + e3)
            ws_ = (e0 * inv, e1 * inv, e2 * inv, e3 * inv)

            def make_agg(nk):
                def body(ci, carry):
                    o = ws_[0] * kv_b[b, 0 * G + t, pl.ds(ci * 16, 16)]
                    for k in range(1, nk):
                        o = o + ws_[k] * kv_b[b, k * G + t, pl.ds(ci * 16, 16)]
                    ag_b[b, t, pl.ds(ci * 16, 16)] = o
                    return carry
                return body

            for v, cond in ((2, ls <= 2), (4, ls >= 3)):
                @pl.when(cond)
                def _(v=v):
                    lax.fori_loop(0, C // 16, make_agg(v), 0, unroll=4)

    def writeback(g, b):
        pltpu.async_copy(ag_b.at[b], agg_out.at[pl.ds(base + g * G, G)], ws[b])

    def wait_writeback(g, b):
        pltpu.make_async_copy(
            ag_b.at[b], agg_out.at[pl.ds(base + g * G, G)], ws[b]).wait()

    issue_group(0, 0)

    def pair_body(j, carry):
        lwin_i = len_v[pl.ds(j * 16, 16)]
        lwinf = lwin_i.astype(jnp.float32)
        for b in range(2):
            g = 2 * j + b

            @pl.when(g + 1 < NGRP)
            def _():
                issue_group(g + 1, (b + 1) % 2)

            wait_group(g, b)

            @pl.when(g >= 2)
            def _():
                wait_writeback(g - 2, b)

            compute_group(g, b, lwinf, lwin_i)
            writeback(g, b)
        return carry

    lax.fori_loop(0, NGRP // 2, pair_body, 0)
    wait_writeback(NGRP - 2, 0)
    wait_writeback(NGRP - 1, 1)


def _dot_t(a, w):
    return lax.dot_general(a, w, (((1,), (1,)), ((), ())),
                           preferred_element_type=jnp.float32)


def _tc_body(x_ref, agg_ref, wup_ref, wmix_ref, out_ref):
    xb = x_ref[...]
    agg = agg_ref[...]
    up = _dot_t(agg.astype(jnp.bfloat16), wup_ref[...])
    upn = up * lax.rsqrt(jnp.mean(up * up, axis=1, keepdims=True) + EPS)
    out_ref[...] = (_dot_t(upn.astype(jnp.bfloat16), wmix_ref[:, :D_UP])
                    + _dot_t(xb.astype(jnp.bfloat16), wmix_ref[:, D_UP:]))


BT = 1024


def _tc_call(x2, agg, wup, wmix):
    return pl.pallas_call(
        _tc_body,
        grid=(TOKENS // BT,),
        in_specs=[
            pl.BlockSpec((BT, C), lambda i: (i, 0)),
            pl.BlockSpec((BT, C), lambda i: (i, 0)),
            pl.BlockSpec((D_UP, C), lambda i: (0, 0)),
            pl.BlockSpec((C, D_UP + C), lambda i: (0, 0)),
        ],
        out_specs=pl.BlockSpec((BT, C), lambda i: (i, 0)),
        out_shape=jax.ShapeDtypeStruct((TOKENS, C), jnp.float32),
    )(x2, agg, wup, wmix)


def kernel(x, token_ids, bounds, kv_weight, w_up, w_mix):
    B, T, _ = x.shape
    flat_ids = token_ids.reshape(B * T).astype(jnp.int32)
    bounds_pad = jnp.pad(bounds.astype(jnp.int32),
                         (0, BOUNDS_PAD - bounds.shape[0]))
    wup_b = w_up.astype(jnp.bfloat16)
    wmix_b = w_mix.astype(jnp.bfloat16)
    x2 = x.reshape(B * T, C)
    agg = _sc_attn(flat_ids, bounds_pad, x2, kv_weight)
    delta = _tc_call(x2, agg, wup_b, wmix_b)
    return delta.reshape(B, T, C)


# speedup vs baseline: 1.5963x; 1.5963x over previous
import functools

import jax
import jax.numpy as jnp
from jax import lax
from jax.experimental import pallas as pl
from jax.experimental.pallas import tpu as pltpu
from jax.experimental.pallas import tpu_sc as plsc

N_EMB = 131072
C = 1024
D_UP = 2048
K_MAX = 4
EPS = 1.1920928955078125e-07

TOKENS = 4096
NC, NS = 2, 16
NW = NC * NS
TPW = TOKENS // NW
CHUNK = 16
NBUF = 4
NCH = (TPW * K_MAX) // CHUNK
LOOKAHEAD = 2
BOUNDS_PAD = 50016

_sc_mesh = plsc.VectorSubcoreMesh(core_axis_name="c", subcore_axis_name="s")


@functools.partial(
    pl.kernel,
    mesh=_sc_mesh,
    out_type=(
        jax.ShapeDtypeStruct((TOKENS, C), jnp.float32),
        jax.ShapeDtypeStruct((TOKENS, C), jnp.float32),
        jax.ShapeDtypeStruct((TOKENS, C), jnp.float32),
        jax.ShapeDtypeStruct((TOKENS, C), jnp.float32),
        jax.ShapeDtypeStruct((TOKENS,), jnp.int32),
    ),
    scratch_types=[
        pltpu.VMEM((TPW,), jnp.int32),
        pltpu.VMEM((TPW,), jnp.int32),
        pltpu.VMEM((TPW,), jnp.int32),
        pltpu.VMEM((TPW,), jnp.int32),
        pltpu.VMEM((K_MAX, TPW), jnp.int32),
        pltpu.VMEM((TPW,), jnp.int32),
        pltpu.VMEM((NBUF, CHUNK, C), jnp.float32),
        pltpu.SemaphoreType.DMA,
        pltpu.SemaphoreType.DMA,
        pltpu.SemaphoreType.DMA,
        pltpu.SemaphoreType.DMA,
        pltpu.SemaphoreType.DMA,
        pltpu.SemaphoreType.DMA,
        pltpu.SemaphoreType.DMA,
        pltpu.SemaphoreType.DMA,
        pltpu.SemaphoreType.DMA,
    ],
)
def _sc_gather(ids_hbm, bounds_hbm, kv_hbm, kv_out0, kv_out1, kv_out2, kv_out3,
               len_out, ids_v, ids1_v, starts_v, ends_v, idx_v, len_v, rows_v,
               sem, g0, g1, g2, g3, w0, w1, w2, w3):
    kv_outs = (kv_out0, kv_out1, kv_out2, kv_out3)
    wid = lax.axis_index("s") * NC + lax.axis_index("c")
    base = wid * TPW
    gsem = (g0, g1, g2, g3)
    wsem = (w0, w1, w2, w3)
    pltpu.sync_copy(ids_hbm.at[pl.ds(base, TPW)], ids_v)
    for j in range(TPW // 16):
        ids1_v[pl.ds(j * 16, 16)] = ids_v[pl.ds(j * 16, 16)] + 1
    cs = pltpu.async_copy(bounds_hbm.at[ids_v], starts_v, sem)
    ce = pltpu.async_copy(bounds_hbm.at[ids1_v], ends_v, sem)
    cs.wait()
    ce.wait()
    for j in range(TPW // 16):
        s16 = starts_v[pl.ds(j * 16, 16)]
        e16 = ends_v[pl.ds(j * 16, 16)]
        len_v[pl.ds(j * 16, 16)] = e16 - s16
        for k in range(K_MAX):
            idx_v[k, pl.ds(j * 16, 16)] = jnp.minimum(s16 + k, N_EMB - 1)
    len_copy = pltpu.async_copy(len_v, len_out.at[pl.ds(base, TPW)], sem)

    hpk = TPW // CHUNK

    def issue_gather(c):
        b = c % NBUF
        k, h = c // hpk, c % hpk
        return pltpu.async_copy(
            kv_hbm.at[idx_v.at[k, pl.ds(h * CHUNK, CHUNK)]], rows_v.at[b], gsem[b])

    def issue_write(c):
        b = c % NBUF
        k, h = c // hpk, c % hpk
        return pltpu.async_copy(
            rows_v.at[b], kv_outs[k].at[pl.ds(base + h * CHUNK, CHUNK)], wsem[b])

    gh = [None] * NCH
    wh = [None] * NCH
    for c in range(NCH + LOOKAHEAD):
        if c < NCH:
            if c >= NBUF:
                wh[c - NBUF].wait()
            gh[c] = issue_gather(c)
        d = c - LOOKAHEAD
        if 0 <= d < NCH:
            gh[d].wait()
            wh[d] = issue_write(d)
    for d in range(NCH - NBUF, NCH):
        wh[d].wait()
    len_copy.wait()


def _dot_t(a, w):
    return lax.dot_general(a, w, (((1,), (1,)), ((), ())),
                           preferred_element_type=jnp.float32)


def _tc_body(x_ref, kv0_ref, kv1_ref, kv2_ref, kv3_ref, len_ref, wup_ref, wmix_ref, out_ref):
    xb = x_ref[...]
    xn = xb * lax.rsqrt(jnp.mean(xb * xb, axis=1, keepdims=True) + EPS)
    lens = len_ref[...]
    kv0, kv1, kv2, kv3 = kv0_ref[...], kv1_ref[...], kv2_ref[...], kv3_ref[...]
    s0 = jnp.sum(kv0 * xn, axis=1, keepdims=True)
    s1 = jnp.sum(kv1 * xn, axis=1, keepdims=True)
    s2 = jnp.sum(kv2 * xn, axis=1, keepdims=True)
    s3 = jnp.sum(kv3 * xn, axis=1, keepdims=True)
    m = s0
    m = jnp.where(lens > 1, jnp.maximum(m, s1), m)
    m = jnp.where(lens > 2, jnp.maximum(m, s2), m)
    m = jnp.where(lens > 3, jnp.maximum(m, s3), m)
    e0 = jnp.exp(s0 - m)
    e1 = jnp.where(lens > 1, jnp.exp(s1 - m), 0.0)
    e2 = jnp.where(lens > 2, jnp.exp(s2 - m), 0.0)
    e3 = jnp.where(lens > 3, jnp.exp(s3 - m), 0.0)
    inv = 1.0 / (e0 + e1 + e2 + e3)
    agg = (e0 * inv) * kv0 + (e1 * inv) * kv1 + (e2 * inv) * kv2 + (e3 * inv) * kv3
    up = _dot_t(agg.astype(jnp.bfloat16), wup_ref[...])
    upn = up * lax.rsqrt(jnp.mean(up * up, axis=1, keepdims=True) + EPS)
    out_ref[...] = (_dot_t(upn.astype(jnp.bfloat16), wmix_ref[:, :D_UP])
                    + _dot_t(xb.astype(jnp.bfloat16), wmix_ref[:, D_UP:]))


BT = 512


def _tc_call(x2, kv_g, lens2, wup, wmix):
    grid = (TOKENS // BT,)
    return pl.pallas_call(
        _tc_body,
        grid=grid,
        in_specs=[
            pl.BlockSpec((BT, C), lambda i: (i, 0)),
            pl.BlockSpec((BT, C), lambda i: (i, 0)),
            pl.BlockSpec((BT, C), lambda i: (i, 0)),
            pl.BlockSpec((BT, C), lambda i: (i, 0)),
            pl.BlockSpec((BT, C), lambda i: (i, 0)),
            pl.BlockSpec((BT, 1), lambda i: (i, 0)),
            pl.BlockSpec((D_UP, C), lambda i: (0, 0)),
            pl.BlockSpec((C, D_UP + C), lambda i: (0, 0)),
        ],
        out_specs=pl.BlockSpec((BT, C), lambda i: (i, 0)),
        out_shape=jax.ShapeDtypeStruct((TOKENS, C), jnp.float32),
    )(x2, kv_g[0], kv_g[1], kv_g[2], kv_g[3], lens2, wup, wmix)


def kernel(x, token_ids, bounds, kv_weight, w_up, w_mix):
    B, T, _ = x.shape
    flat_ids = token_ids.reshape(B * T).astype(jnp.int32)
    bounds_pad = jnp.pad(bounds.astype(jnp.int32),
                         (0, BOUNDS_PAD - bounds.shape[0]))
    k0, k1, k2, k3, lens = _sc_gather(flat_ids, bounds_pad, kv_weight)
    kv_g = (k0, k1, k2, k3)
    delta = _tc_call(
        x.reshape(B * T, C),
        kv_g,
        lens.reshape(B * T, 1),
        w_up.astype(jnp.bfloat16),
        w_mix.astype(jnp.bfloat16),
    )
    return delta.reshape(B, T, C)


# speedup vs baseline: 1.6137x; 1.0109x over previous
import functools

import jax
import jax.numpy as jnp
from jax import lax
from jax.experimental import pallas as pl
from jax.experimental.pallas import tpu as pltpu
from jax.experimental.pallas import tpu_sc as plsc

N_EMB = 131072
C = 1024
D_UP = 2048
K_MAX = 4
EPS = 1.1920928955078125e-07

TOKENS = 4096
NC, NS = 2, 16
NW = NC * NS
TPW = TOKENS // NW
CHUNK = 32
NBUF = 3
NCH = (TPW * K_MAX) // CHUNK
LOOKAHEAD = 2
BOUNDS_PAD = 50016

_sc_mesh = plsc.VectorSubcoreMesh(core_axis_name="c", subcore_axis_name="s")


@functools.partial(
    pl.kernel,
    mesh=_sc_mesh,
    out_type=(
        jax.ShapeDtypeStruct((K_MAX, TOKENS, C), jnp.float32),
        jax.ShapeDtypeStruct((TOKENS,), jnp.int32),
    ),
    scratch_types=[
        pltpu.VMEM((TPW,), jnp.int32),
        pltpu.VMEM((TPW,), jnp.int32),
        pltpu.VMEM((TPW,), jnp.int32),
        pltpu.VMEM((TPW,), jnp.int32),
        pltpu.VMEM((K_MAX, TPW), jnp.int32),
        pltpu.VMEM((TPW,), jnp.int32),
        pltpu.VMEM((NBUF, CHUNK, C), jnp.float32),
        pltpu.SemaphoreType.DMA,
        pltpu.SemaphoreType.DMA,
        pltpu.SemaphoreType.DMA,
        pltpu.SemaphoreType.DMA,
        pltpu.SemaphoreType.DMA,
        pltpu.SemaphoreType.DMA,
        pltpu.SemaphoreType.DMA,
        pltpu.SemaphoreType.DMA,
        pltpu.SemaphoreType.DMA,
    ],
)
def _sc_gather(ids_hbm, bounds_hbm, kv_hbm, kv_out, len_out,
               ids_v, ids1_v, starts_v, ends_v, idx_v, len_v, rows_v,
               sem, g0, g1, g2, g3, w0, w1, w2, w3):
    wid = lax.axis_index("s") * NC + lax.axis_index("c")
    base = wid * TPW
    gsem = (g0, g1, g2, g3)
    wsem = (w0, w1, w2, w3)
    pltpu.sync_copy(ids_hbm.at[pl.ds(base, TPW)], ids_v)
    for j in range(TPW // 16):
        ids1_v[pl.ds(j * 16, 16)] = ids_v[pl.ds(j * 16, 16)] + 1
    cs = pltpu.async_copy(bounds_hbm.at[ids_v], starts_v, sem)
    ce = pltpu.async_copy(bounds_hbm.at[ids1_v], ends_v, sem)
    cs.wait()
    ce.wait()
    for j in range(TPW // 16):
        s16 = starts_v[pl.ds(j * 16, 16)]
        e16 = ends_v[pl.ds(j * 16, 16)]
        len_v[pl.ds(j * 16, 16)] = e16 - s16
        for k in range(K_MAX):
            idx_v[k, pl.ds(j * 16, 16)] = jnp.minimum(s16 + k, N_EMB - 1)
    len_copy = pltpu.async_copy(len_v, len_out.at[pl.ds(base, TPW)], sem)

    hpk = TPW // CHUNK

    def issue_gather(c):
        b = c % NBUF
        k, h = c // hpk, c % hpk
        return pltpu.async_copy(
            kv_hbm.at[idx_v.at[k, pl.ds(h * CHUNK, CHUNK)]], rows_v.at[b], gsem[b])

    def issue_write(c):
        b = c % NBUF
        k, h = c // hpk, c % hpk
        return pltpu.async_copy(
            rows_v.at[b], kv_out.at[k, pl.ds(base + h * CHUNK, CHUNK)], wsem[b])

    gh = [None] * NCH
    wh = [None] * NCH
    for c in range(NCH + LOOKAHEAD):
        if c < NCH:
            if c >= NBUF:
                wh[c - NBUF].wait()
            gh[c] = issue_gather(c)
        d = c - LOOKAHEAD
        if 0 <= d < NCH:
            gh[d].wait()
            wh[d] = issue_write(d)
    for d in range(NCH - NBUF, NCH):
        wh[d].wait()
    len_copy.wait()


def _dot_t(a, w):
    return lax.dot_general(a, w, (((1,), (1,)), ((), ())),
                           preferred_element_type=jnp.float32)


def _tc_body(x_ref, kv_ref, len_ref, wup_ref, wmix_ref, out_ref):
    xb = x_ref[...]
    xn = xb * lax.rsqrt(jnp.mean(xb * xb, axis=1, keepdims=True) + EPS)
    lens = len_ref[...]
    kv0, kv1, kv2, kv3 = kv_ref[0], kv_ref[1], kv_ref[2], kv_ref[3]
    s0 = jnp.sum(kv0 * xn, axis=1, keepdims=True)
    s1 = jnp.sum(kv1 * xn, axis=1, keepdims=True)
    s2 = jnp.sum(kv2 * xn, axis=1, keepdims=True)
    s3 = jnp.sum(kv3 * xn, axis=1, keepdims=True)
    m = s0
    m = jnp.where(lens > 1, jnp.maximum(m, s1), m)
    m = jnp.where(lens > 2, jnp.maximum(m, s2), m)
    m = jnp.where(lens > 3, jnp.maximum(m, s3), m)
    e0 = jnp.exp(s0 - m)
    e1 = jnp.where(lens > 1, jnp.exp(s1 - m), 0.0)
    e2 = jnp.where(lens > 2, jnp.exp(s2 - m), 0.0)
    e3 = jnp.where(lens > 3, jnp.exp(s3 - m), 0.0)
    inv = 1.0 / (e0 + e1 + e2 + e3)
    agg = (e0 * inv) * kv0 + (e1 * inv) * kv1 + (e2 * inv) * kv2 + (e3 * inv) * kv3
    up = _dot_t(agg.astype(jnp.bfloat16), wup_ref[...])
    upn = up * lax.rsqrt(jnp.mean(up * up, axis=1, keepdims=True) + EPS)
    out_ref[...] = (_dot_t(upn.astype(jnp.bfloat16), wmix_ref[:, :D_UP])
                    + _dot_t(xb.astype(jnp.bfloat16), wmix_ref[:, D_UP:]))


BT = 512


def _tc_call(x2, kv_g, lens2, wup, wmix):
    grid = (TOKENS // BT,)
    return pl.pallas_call(
        _tc_body,
        grid=grid,
        in_specs=[
            pl.BlockSpec((BT, C), lambda i: (i, 0)),
            pl.BlockSpec((K_MAX, BT, C), lambda i: (0, i, 0)),
            pl.BlockSpec((BT, 1), lambda i: (i, 0)),
            pl.BlockSpec((D_UP, C), lambda i: (0, 0)),
            pl.BlockSpec((C, D_UP + C), lambda i: (0, 0)),
        ],
        out_specs=pl.BlockSpec((BT, C), lambda i: (i, 0)),
        out_shape=jax.ShapeDtypeStruct((TOKENS, C), jnp.float32),
    )(x2, kv_g, lens2, wup, wmix)


def kernel(x, token_ids, bounds, kv_weight, w_up, w_mix):
    B, T, _ = x.shape
    flat_ids = token_ids.reshape(B * T).astype(jnp.int32)
    bounds_pad = jnp.pad(bounds.astype(jnp.int32),
                         (0, BOUNDS_PAD - bounds.shape[0]))
    kv_g, lens = _sc_gather(flat_ids, bounds_pad, kv_weight)
    delta = _tc_call(
        x.reshape(B * T, C),
        kv_g,
        lens.reshape(B * T, 1),
        w_up.astype(jnp.bfloat16),
        w_mix.astype(jnp.bfloat16),
    )
    return delta.reshape(B, T, C)
